# Initial kernel scaffold; baseline (speedup 1.0000x reference)
import jax
import jax.numpy as jnp
from jax.experimental import pallas as pl


def kernel(x, pos_table):
    raise NotImplementedError("write your pallas kernel here")


# speedup vs baseline: 2.5116x; 2.5116x over previous
import jax
import jax.numpy as jnp
from jax.experimental import pallas as pl


_BLOCK_ROWS = 512


def _add_kernel(x_ref, pos_ref, out_ref):
    out_ref[...] = x_ref[...] + pos_ref[...]


def kernel(x, pos_table):
    batch, seq_len, d_model = x.shape
    xf = x.reshape(batch * seq_len, d_model)
    n_blocks = (batch * seq_len) // _BLOCK_ROWS
    s_blocks = seq_len // _BLOCK_ROWS

    out = pl.pallas_call(
        _add_kernel,
        grid=(n_blocks,),
        in_specs=[
            pl.BlockSpec((_BLOCK_ROWS, d_model), lambda i: (i, 0)),
            pl.BlockSpec((_BLOCK_ROWS, d_model), lambda i: (i % s_blocks, 0)),
        ],
        out_specs=pl.BlockSpec((_BLOCK_ROWS, d_model), lambda i: (i, 0)),
        out_shape=jax.ShapeDtypeStruct((batch * seq_len, d_model), x.dtype),
    )(xf, pos_table)
    return out.reshape(batch, seq_len, d_model)


# speedup vs baseline: 2.5780x; 1.0264x over previous
import jax
import jax.numpy as jnp
from jax.experimental import pallas as pl


_BLOCK_ROWS = 1024


def _add_kernel(x_ref, pos_ref, out_ref):
    out_ref[...] = x_ref[...] + pos_ref[...]


def kernel(x, pos_table):
    batch, seq_len, d_model = x.shape
    xf = x.reshape(batch * seq_len, d_model)
    n_blocks = (batch * seq_len) // _BLOCK_ROWS
    s_blocks = seq_len // _BLOCK_ROWS

    out = pl.pallas_call(
        _add_kernel,
        grid=(n_blocks,),
        in_specs=[
            pl.BlockSpec((_BLOCK_ROWS, d_model), lambda i: (i, 0)),
            pl.BlockSpec((_BLOCK_ROWS, d_model), lambda i: (i % s_blocks, 0)),
        ],
        out_specs=pl.BlockSpec((_BLOCK_ROWS, d_model), lambda i: (i, 0)),
        out_shape=jax.ShapeDtypeStruct((batch * seq_len, d_model), x.dtype),
    )(xf, pos_table)
    return out.reshape(batch, seq_len, d_model)
